# Initial kernel scaffold; baseline (speedup 1.0000x reference)
import jax
import jax.numpy as jnp
from jax import lax
from jax.experimental import pallas as pl
from jax.experimental.pallas import tpu as pltpu
from jax.experimental.pallas import tpu_sc as plsc

_NUM_HEADS = 16
_EMBED = 3969
_S = 1024
_NC = 2
_NS = 16
_L = 16
_NW = _NC * _NS
_ROWS_PER_W = _S // _NW


def _gather_body(table_hbm, idx_hbm, out_hbm, table_v, idx_v, out_v):
  wid = lax.axis_index("s") * _NC + lax.axis_index("c")
  pltpu.sync_copy(table_hbm, table_v)

  def do_row(r, carry):
    row = wid * _ROWS_PER_W + r
    pltpu.sync_copy(idx_hbm.at[row], idx_v)

    def do_group(g, c):
      vi = idx_v[pl.ds(g * _L, _L)]
      for h in range(_NUM_HEADS):
        out_v[h, pl.ds(g * _L, _L)] = plsc.load_gather(
            table_v, [vi + h * _EMBED])
      return c

    lax.fori_loop(0, _S // _L, do_group, 0)
    for h in range(_NUM_HEADS):
      pltpu.sync_copy(out_v.at[h], out_hbm.at[h, row])
    return carry

  lax.fori_loop(0, _ROWS_PER_W, do_row, 0)


@jax.jit
def kernel(attn_rpe_index, relative_position_bias_table):
  idx = attn_rpe_index.astype(jnp.int32)
  table_flat = relative_position_bias_table.reshape(-1)
  mesh = plsc.VectorSubcoreMesh(
      core_axis_name="c", subcore_axis_name="s",
      num_cores=_NC, num_subcores=_NS)
  out = pl.kernel(
      _gather_body,
      out_type=jax.ShapeDtypeStruct((_NUM_HEADS, _S, _S), jnp.float32),
      mesh=mesh,
      scratch_types=[
          pltpu.VMEM((_NUM_HEADS * _EMBED,), jnp.float32),
          pltpu.VMEM((_S,), jnp.int32),
          pltpu.VMEM((_NUM_HEADS, _S), jnp.float32),
      ],
  )(table_flat, idx)
  return out[None]


# speedup vs baseline: 16.5246x; 16.5246x over previous
import jax
import jax.numpy as jnp
from jax import lax
from jax.experimental import pallas as pl
from jax.experimental.pallas import tpu as pltpu
from jax.experimental.pallas import tpu_sc as plsc

_NUM_HEADS = 16
_EMBED = 3969
_S = 1024
_NC = 2
_NS = 16
_L = 16
_NW = _NC * _NS
_ROWS_PER_W = _S // _NW


def _gather_body(table_hbm, idx_hbm, out_hbm, table_v, idx_v, out_v):
  wid = lax.axis_index("s") * _NC + lax.axis_index("c")
  pltpu.sync_copy(table_hbm, table_v)

  def do_row(r, carry):
    row = wid * _ROWS_PER_W + r
    pltpu.sync_copy(idx_hbm.at[row], idx_v)

    def do_group(g, c):
      vi = idx_v[pl.ds(g * _L, _L)]
      for h in range(_NUM_HEADS):
        out_v[h, pl.ds(g * _L, _L)] = plsc.load_gather(
            table_v, [vi + h * _EMBED])
      return c

    lax.fori_loop(0, _S // _L, do_group, 0)
    for h in range(_NUM_HEADS):
      pltpu.sync_copy(out_v.at[h], out_hbm.at[h, row])
    return carry

  lax.fori_loop(0, _ROWS_PER_W, do_row, 0)


@jax.jit
def kernel(attn_rpe_index, relative_position_bias_table):
  idx = attn_rpe_index.astype(jnp.int32)
  table_flat = relative_position_bias_table.reshape(-1)
  mesh = plsc.VectorSubcoreMesh(
      core_axis_name="c", subcore_axis_name="s",
      num_cores=_NC, num_subcores=_NS)
  out = pl.kernel(
      _gather_body,
      out_type=jax.ShapeDtypeStruct((_NUM_HEADS, _S, _S), jnp.float32),
      mesh=mesh,
      scratch_types=[
          pltpu.VMEM((_NUM_HEADS * _EMBED,), jnp.float32),
          pltpu.VMEM((_S,), jnp.int32),
          pltpu.VMEM((_NUM_HEADS, _S), jnp.float32),
      ],
      compiler_params=pltpu.CompilerParams(needs_layout_passes=False),
  )(table_flat, idx)
  return out[None]


# speedup vs baseline: 52.8116x; 3.1959x over previous
import jax
import jax.numpy as jnp
from jax import lax
from jax.experimental import pallas as pl
from jax.experimental.pallas import tpu as pltpu
from jax.experimental.pallas import tpu_sc as plsc

_NUM_HEADS = 16
_EMBED = 3969
_S = 1024
_NC = 2
_NS = 16
_L = 16
_NW = _NC * _NS
_ROWS_PER_W = _S // _NW
_GROUPS = _S // _L


def _gather_body(table_hbm, idx_hbm, out_hbm,
                 table_v, idx_v, out0_v, out1_v, osem0, osem1):
  wid = lax.axis_index("s") * _NC + lax.axis_index("c")
  row0 = wid * _ROWS_PER_W
  pltpu.sync_copy(table_hbm, table_v)
  pltpu.sync_copy(idx_hbm.at[pl.ds(row0, _ROWS_PER_W)], idx_v)

  obufs = (out0_v, out1_v)
  osems = (osem0, osem1)

  def do_pair(k, carry):
    for b in range(2):
      r = 2 * k + b
      obuf, osem = obufs[b], osems[b]

      @pl.when(k > 0)
      def _wait():
        for h in range(_NUM_HEADS):
          pltpu.make_async_copy(obuf.at[h], out_hbm.at[h, row0], osem).wait()

      def do_group(g, obuf=obuf, r=r):
        vi = idx_v[r, pl.ds(g * _L, _L)]
        for h in range(_NUM_HEADS):
          obuf[h, pl.ds(g * _L, _L)] = plsc.load_gather(
              table_v, [vi + h * _EMBED])

      plsc.parallel_loop(0, _GROUPS, unroll=2)(do_group)
      for h in range(_NUM_HEADS):
        pltpu.async_copy(obuf.at[h], out_hbm.at[h, row0 + r], osem)
    return carry

  lax.fori_loop(0, _ROWS_PER_W // 2, do_pair, 0)
  for h in range(_NUM_HEADS):
    pltpu.make_async_copy(out0_v.at[h], out_hbm.at[h, row0], osem0).wait()
    pltpu.make_async_copy(out1_v.at[h], out_hbm.at[h, row0], osem1).wait()


@jax.jit
def kernel(attn_rpe_index, relative_position_bias_table):
  idx = attn_rpe_index.astype(jnp.int32)
  table_flat = relative_position_bias_table.reshape(-1)
  mesh = plsc.VectorSubcoreMesh(
      core_axis_name="c", subcore_axis_name="s",
      num_cores=_NC, num_subcores=_NS)
  out = pl.kernel(
      _gather_body,
      out_type=jax.ShapeDtypeStruct((_NUM_HEADS, _S, _S), jnp.float32),
      mesh=mesh,
      scratch_types=[
          pltpu.VMEM((_NUM_HEADS * _EMBED,), jnp.float32),
          pltpu.VMEM((_ROWS_PER_W, _S), jnp.int32),
          pltpu.VMEM((_NUM_HEADS, _S), jnp.float32),
          pltpu.VMEM((_NUM_HEADS, _S), jnp.float32),
          pltpu.SemaphoreType.DMA,
          pltpu.SemaphoreType.DMA,
      ],
      compiler_params=pltpu.CompilerParams(needs_layout_passes=False),
  )(table_flat, idx)
  return out[None]
